# baseline (speedup 1.0000x reference)
import functools

import jax
import jax.numpy as jnp
from jax import lax
from jax.experimental import pallas as pl
from jax.experimental.pallas import tpu as pltpu
from jax.experimental.pallas import tpu_sc as plsc

B, C_IN, C_OUT, C_COORD, N, K = 2, 256, 256, 3, 4096, 16
NB1 = 256
NB3 = 256
CPAD = 128
HALF = C_OUT // 2
TAB = C_OUT // 2 + CPAD
BIGF = 3e38


def _pack16(x):
    bits = lax.bitcast_convert_type(x, jnp.int32)
    return lax.shift_right_arithmetic(bits + 0x8000, 16) & 0xFFFF


def _unpack_lo(w):
    return lax.bitcast_convert_type(lax.shift_left(w, 16), jnp.float32)


def _unpack_hi(w):
    return lax.bitcast_convert_type(w & jnp.int32(-65536), jnp.float32)


def _stage1a_body(fT_blk_ref, crd_ref, W_phi_ref, b_phi_ref,
                  W_psi_ref, b_psi_ref, W_alpha_ref, b_alpha_ref,
                  phiT_ref, alphaT_ref, tab_ref):
    f_blk = fT_blk_ref[0]
    dims = (((1,), (1,)), ((), ()))
    phiT_ref[0] = lax.dot_general(
        f_blk, W_phi_ref[...], dims,
        preferred_element_type=jnp.float32) + b_phi_ref[...]
    alphaT_ref[0] = lax.dot_general(
        f_blk, W_alpha_ref[...], dims,
        preferred_element_type=jnp.float32) + b_alpha_ref[...]
    psi = lax.dot_general(
        f_blk, W_psi_ref[...], dims,
        preferred_element_type=jnp.float32) + b_psi_ref[...]
    tab_ref[0, :, :HALF] = (_pack16(psi[:, :HALF])
                            | lax.shift_left(_pack16(psi[:, HALF:]), 16))
    tab_ref[0, :, HALF:] = _pack16(crd_ref[0])


def _stage1a(fT, crdT, W_phi, b_phi, W_psi, b_psi, W_alpha, b_alpha):
    nblk = N // NB1
    grid = (1, nblk)
    wspec = pl.BlockSpec((C_OUT, C_IN), lambda b, i: (0, 0))
    bspec = pl.BlockSpec((1, C_OUT), lambda b, i: (0, 0))
    outT = jax.ShapeDtypeStruct((1, N, C_OUT), jnp.float32)
    return pl.pallas_call(
        _stage1a_body,
        grid=grid,
        in_specs=[
            pl.BlockSpec((1, NB1, C_IN), lambda b, i: (b, i, 0)),
            pl.BlockSpec((1, NB1, CPAD), lambda b, i: (b, i, 0)),
            wspec, bspec, wspec, bspec, wspec, bspec,
        ],
        out_specs=[
            pl.BlockSpec((1, NB1, C_OUT), lambda b, i: (b, i, 0)),
            pl.BlockSpec((1, NB1, C_OUT), lambda b, i: (b, i, 0)),
            pl.BlockSpec((1, NB1, TAB), lambda b, i: (b, i, 0)),
        ],
        out_shape=[outT, outT,
                   jax.ShapeDtypeStruct((1, N, TAB), jnp.int32)],
        compiler_params=pltpu.CompilerParams(
            dimension_semantics=("parallel", "parallel")),
    )(fT, crdT, W_phi, b_phi.reshape(1, C_OUT), W_psi,
      b_psi.reshape(1, C_OUT), W_alpha, b_alpha.reshape(1, C_OUT))


def _stage1b_body(fT_all_ref, fT_blk_ref, idx_ref):
    f_all = fT_all_ref[0]
    f_blk = fT_blk_ref[0]
    dims = (((1,), (1,)), ((), ()))
    inner = lax.dot_general(f_all, f_blk, dims,
                            preferred_element_type=jnp.float32)
    sq = jnp.sum(f_all * f_all, axis=1, keepdims=True)
    dist = sq - (inner + inner)

    row_iota = lax.broadcasted_iota(jnp.int32, (N, NB1), 0)
    for k in range(K):
        sel = jnp.argmin(dist, axis=0).astype(jnp.int32)
        idx_ref[0, k] = sel
        dist = jnp.where(row_iota == sel[None, :], BIGF, dist)


def _stage1b(fT, fT_chunk):
    nch = fT_chunk.shape[1]
    nblk = nch // NB1
    grid = (1, nblk)
    return pl.pallas_call(
        _stage1b_body,
        grid=grid,
        in_specs=[
            pl.BlockSpec((1, N, C_IN), lambda b, i: (b, 0, 0)),
            pl.BlockSpec((1, NB1, C_IN), lambda b, i: (b, i, 0)),
        ],
        out_specs=pl.BlockSpec((1, K, NB1), lambda b, i: (b, 0, i)),
        out_shape=jax.ShapeDtypeStruct((1, K, nch), jnp.int32),
        compiler_params=pltpu.CompilerParams(
            dimension_semantics=("parallel", "parallel")),
    )(fT, fT_chunk)


_GW = 128


def _sc_gather(table, idx_flat):
    n_idx = idx_flat.shape[1]
    mesh = plsc.VectorSubcoreMesh(core_axis_name="c", subcore_axis_name="s")

    @functools.partial(
        pl.kernel,
        out_type=jax.ShapeDtypeStruct((n_idx, TAB), jnp.int32),
        mesh=mesh,
    )
    def gather_kernel(tab_hbm, i_hbm, out_hbm):
        def body(i_vmem, o_vmem):
            pltpu.sync_copy(tab_hbm.at[i_vmem.at[0]], o_vmem)

        pltpu.emit_pipeline(
            body,
            grid=(n_idx // _GW,),
            in_specs=[pl.BlockSpec((1, _GW), index_map=lambda i: (0, i))],
            out_specs=[pl.BlockSpec((_GW, TAB), index_map=lambda i: (i, 0))],
            core_axis_name=("c", "s"),
            dimension_semantics=(pltpu.PARALLEL,),
        )(i_hbm, out_hbm)

    return gather_kernel(table, idx_flat)


def _stage2_body(nbr_ref, crdc_ref, phi_ref, alpha_ref,
                 W_t1_ref, b_t1_ref, W_t2_ref, b_t2_ref,
                 W_g1_ref, b_g1_ref, W_g2_ref, b_g2_ref, out_ref):
    dims = (((1,), (1,)), ((), ()))
    M = K * NB3

    nbr = nbr_ref[0]
    psi_w = nbr[:, :, :HALF]
    psi_nbr = jnp.concatenate(
        [_unpack_lo(psi_w), _unpack_hi(psi_w)], axis=-1)
    c_nbr = _unpack_lo(nbr[:, :, HALF:])
    c_ctr = crdc_ref[0]

    bf = jnp.bfloat16
    cs = (c_ctr[None, :, :] - c_nbr).reshape(M, CPAD)
    h = lax.dot_general(cs.astype(bf), W_t1_ref[...].astype(bf), dims,
                        preferred_element_type=jnp.float32) + b_t1_ref[...]
    h = jnp.maximum(h, 0.0)
    delta = lax.dot_general(h.astype(bf), W_t2_ref[...].astype(bf), dims,
                            preferred_element_type=jnp.float32) + b_t2_ref[...]

    ginp = (phi_ref[0][None, :, :] - psi_nbr).reshape(M, C_OUT) + delta
    g = lax.dot_general(ginp.astype(bf), W_g1_ref[...].astype(bf), dims,
                        preferred_element_type=jnp.float32) + b_g1_ref[...]
    g = jnp.maximum(g, 0.0)
    gamma = lax.dot_general(g.astype(bf), W_g2_ref[...].astype(bf), dims,
                            preferred_element_type=jnp.float32) + b_g2_ref[...]

    gamma = gamma.reshape(K, NB3, C_OUT)
    delta = delta.reshape(K, NB3, C_OUT)
    m = jnp.max(gamma, axis=0)
    e = jnp.exp(gamma - m[None, :, :])
    s = jnp.sum(e, axis=0)
    acc = jnp.sum(e * delta, axis=0)
    out_ref[0] = alpha_ref[0] + acc / s


def _stage2(nbr, crdT, phiT, alphaT,
            W_t1p, b_t1, W_t2, b_t2, W_g1, b_g1, W_g2, b_g2):
    npts = nbr.shape[2]
    nblk = npts // NB3
    grid = (1, nblk)
    wspec = pl.BlockSpec((C_OUT, C_OUT), lambda b, i: (0, 0))
    bspec = pl.BlockSpec((1, C_OUT), lambda b, i: (0, 0))
    return pl.pallas_call(
        _stage2_body,
        grid=grid,
        in_specs=[
            pl.BlockSpec((1, K, NB3, TAB), lambda b, i: (b, 0, i, 0)),
            pl.BlockSpec((1, NB3, CPAD), lambda b, i: (b, i, 0)),
            pl.BlockSpec((1, NB3, C_OUT), lambda b, i: (b, i, 0)),
            pl.BlockSpec((1, NB3, C_OUT), lambda b, i: (b, i, 0)),
            pl.BlockSpec((C_OUT, CPAD), lambda b, i: (0, 0)), bspec,
            wspec, bspec, wspec, bspec, wspec, bspec,
        ],
        out_specs=pl.BlockSpec((1, NB3, C_OUT), lambda b, i: (b, i, 0)),
        out_shape=jax.ShapeDtypeStruct((1, npts, C_OUT), jnp.float32),
        compiler_params=pltpu.CompilerParams(
            dimension_semantics=("parallel", "parallel")),
    )(nbr, crdT, phiT, alphaT,
      W_t1p, b_t1.reshape(1, C_OUT), W_t2, b_t2.reshape(1, C_OUT),
      W_g1, b_g1.reshape(1, C_OUT), W_g2, b_g2.reshape(1, C_OUT))


def kernel(features, coords, W_phi, b_phi, W_psi, b_psi, W_alpha, b_alpha,
           W_g1, b_g1, W_g2, b_g2, W_t1, b_t1, W_t2, b_t2):
    fT = jnp.transpose(features, (0, 2, 1))
    crdT = jnp.pad(jnp.transpose(coords, (0, 2, 1)),
                   ((0, 0), (0, 0), (0, CPAD - C_COORD)))
    W_t1p = jnp.pad(W_t1, ((0, 0), (0, CPAD - C_COORD)))

    NCH = N // 4
    outs = []
    for b in range(B):
        phiT, alphaT, table = _stage1a(
            fT[b:b + 1], crdT[b:b + 1],
            W_phi, b_phi, W_psi, b_psi, W_alpha, b_alpha)
        chunks = []
        for c in range(N // NCH):
            lo, hi = c * NCH, (c + 1) * NCH
            idx_c = _stage1b(fT[b:b + 1], fT[b:b + 1, lo:hi])
            nbr = _gather_rows(table.reshape(N, TAB),
                               idx_c.reshape(1, K * NCH))
            chunks.append(_stage2(
                nbr.reshape(1, K, NCH, TAB), crdT[b:b + 1, lo:hi],
                phiT[:, lo:hi], alphaT[:, lo:hi],
                W_t1p, b_t1, W_t2, b_t2, W_g1, b_g1, W_g2, b_g2))
        outs.append(jnp.concatenate(chunks, axis=1))
    outT = jnp.concatenate(outs, axis=0)
    return jnp.transpose(outT, (0, 2, 1))


_gather_rows = _sc_gather


# speedup vs baseline: 1.0119x; 1.0119x over previous
import functools

import jax
import jax.numpy as jnp
from jax import lax
from jax.experimental import pallas as pl
from jax.experimental.pallas import tpu as pltpu
from jax.experimental.pallas import tpu_sc as plsc

B, C_IN, C_OUT, C_COORD, N, K = 2, 256, 256, 3, 4096, 16
NB1 = 256
NB3 = 256
CPAD = 128
HALF = C_OUT // 2
TAB = C_OUT // 2 + CPAD
BIGF = 3e38


def _pack16(x):
    bits = lax.bitcast_convert_type(x, jnp.int32)
    return lax.shift_right_arithmetic(bits + 0x8000, 16) & 0xFFFF


def _unpack_lo(w):
    return lax.bitcast_convert_type(lax.shift_left(w, 16), jnp.float32)


def _unpack_hi(w):
    return lax.bitcast_convert_type(w & jnp.int32(-65536), jnp.float32)


def _stage1a_body(fT_blk_ref, crd_ref, W_phi_ref, b_phi_ref,
                  W_psi_ref, b_psi_ref, W_alpha_ref, b_alpha_ref,
                  phiT_ref, alphaT_ref, tab_ref):
    f_blk = fT_blk_ref[0]
    dims = (((1,), (1,)), ((), ()))
    phiT_ref[0] = lax.dot_general(
        f_blk, W_phi_ref[...], dims,
        preferred_element_type=jnp.float32) + b_phi_ref[...]
    alphaT_ref[0] = lax.dot_general(
        f_blk, W_alpha_ref[...], dims,
        preferred_element_type=jnp.float32) + b_alpha_ref[...]
    psi = lax.dot_general(
        f_blk, W_psi_ref[...], dims,
        preferred_element_type=jnp.float32) + b_psi_ref[...]
    tab_ref[0, :, :HALF] = (_pack16(psi[:, :HALF])
                            | lax.shift_left(_pack16(psi[:, HALF:]), 16))
    tab_ref[0, :, HALF:] = _pack16(crd_ref[0])


def _stage1a(fT, crdT, W_phi, b_phi, W_psi, b_psi, W_alpha, b_alpha):
    nblk = N // NB1
    grid = (1, nblk)
    wspec = pl.BlockSpec((C_OUT, C_IN), lambda b, i: (0, 0))
    bspec = pl.BlockSpec((1, C_OUT), lambda b, i: (0, 0))
    outT = jax.ShapeDtypeStruct((1, N, C_OUT), jnp.float32)
    return pl.pallas_call(
        _stage1a_body,
        grid=grid,
        in_specs=[
            pl.BlockSpec((1, NB1, C_IN), lambda b, i: (b, i, 0)),
            pl.BlockSpec((1, NB1, CPAD), lambda b, i: (b, i, 0)),
            wspec, bspec, wspec, bspec, wspec, bspec,
        ],
        out_specs=[
            pl.BlockSpec((1, NB1, C_OUT), lambda b, i: (b, i, 0)),
            pl.BlockSpec((1, NB1, C_OUT), lambda b, i: (b, i, 0)),
            pl.BlockSpec((1, NB1, TAB), lambda b, i: (b, i, 0)),
        ],
        out_shape=[outT, outT,
                   jax.ShapeDtypeStruct((1, N, TAB), jnp.int32)],
        compiler_params=pltpu.CompilerParams(
            dimension_semantics=("parallel", "parallel")),
    )(fT, crdT, W_phi, b_phi.reshape(1, C_OUT), W_psi,
      b_psi.reshape(1, C_OUT), W_alpha, b_alpha.reshape(1, C_OUT))


def _stage1b_body(fT_all_ref, fT_blk_ref, idx_ref):
    f_all = fT_all_ref[0]
    f_blk = fT_blk_ref[0]
    dims = (((1,), (1,)), ((), ()))
    inner = lax.dot_general(f_all, f_blk, dims,
                            preferred_element_type=jnp.float32)
    sq = jnp.sum(f_all * f_all, axis=1, keepdims=True)
    dist = sq - (inner + inner)

    row_iota = lax.broadcasted_iota(jnp.int32, (N, NB1), 0)
    for k in range(K):
        sel = jnp.argmin(dist, axis=0).astype(jnp.int32)
        idx_ref[0, k] = sel
        dist = jnp.where(row_iota == sel[None, :], BIGF, dist)


def _stage1b(fT, fT_chunk):
    nch = fT_chunk.shape[1]
    nblk = nch // NB1
    grid = (1, nblk)
    return pl.pallas_call(
        _stage1b_body,
        grid=grid,
        in_specs=[
            pl.BlockSpec((1, N, C_IN), lambda b, i: (b, 0, 0)),
            pl.BlockSpec((1, NB1, C_IN), lambda b, i: (b, i, 0)),
        ],
        out_specs=pl.BlockSpec((1, K, NB1), lambda b, i: (b, 0, i)),
        out_shape=jax.ShapeDtypeStruct((1, K, nch), jnp.int32),
        compiler_params=pltpu.CompilerParams(
            dimension_semantics=("parallel", "parallel")),
    )(fT, fT_chunk)


_GW = 128


def _sc_gather(table, idx_flat):
    n_idx = idx_flat.shape[1]
    mesh = plsc.VectorSubcoreMesh(core_axis_name="c", subcore_axis_name="s")

    @functools.partial(
        pl.kernel,
        out_type=jax.ShapeDtypeStruct((n_idx, TAB), jnp.int32),
        mesh=mesh,
    )
    def gather_kernel(tab_hbm, i_hbm, out_hbm):
        def body(i_vmem, o_vmem):
            pltpu.sync_copy(tab_hbm.at[i_vmem.at[0]], o_vmem)

        pltpu.emit_pipeline(
            body,
            grid=(n_idx // _GW,),
            in_specs=[pl.BlockSpec((1, _GW), index_map=lambda i: (0, i))],
            out_specs=[pl.BlockSpec((_GW, TAB), index_map=lambda i: (i, 0))],
            core_axis_name=("c", "s"),
            dimension_semantics=(pltpu.PARALLEL,),
        )(i_hbm, out_hbm)

    return gather_kernel(table, idx_flat)


def _stage2_body(nbr_ref, crdc_ref, phi_ref, alpha_ref,
                 W_t1_ref, b_t1_ref, W_t2_ref, b_t2_ref,
                 W_g1_ref, b_g1_ref, W_g2_ref, b_g2_ref, out_ref):
    dims = (((1,), (1,)), ((), ()))
    M = K * NB3

    nbr = nbr_ref[0]
    psi_w = nbr[:, :, :HALF]
    psi_nbr = jnp.concatenate(
        [_unpack_lo(psi_w), _unpack_hi(psi_w)], axis=-1)
    c_nbr = _unpack_lo(nbr[:, :, HALF:])
    c_ctr = crdc_ref[0]

    bf = jnp.bfloat16
    cs = (c_ctr[None, :, :] - c_nbr).reshape(M, CPAD)
    h = lax.dot_general(cs.astype(bf), W_t1_ref[...].astype(bf), dims,
                        preferred_element_type=jnp.float32) + b_t1_ref[...]
    h = jnp.maximum(h, 0.0)
    delta = lax.dot_general(h.astype(bf), W_t2_ref[...].astype(bf), dims,
                            preferred_element_type=jnp.float32) + b_t2_ref[...]

    ginp = (phi_ref[0][None, :, :] - psi_nbr).reshape(M, C_OUT) + delta
    g = lax.dot_general(ginp.astype(bf), W_g1_ref[...].astype(bf), dims,
                        preferred_element_type=jnp.float32) + b_g1_ref[...]
    g = jnp.maximum(g, 0.0)
    gamma = lax.dot_general(g.astype(bf), W_g2_ref[...].astype(bf), dims,
                            preferred_element_type=jnp.float32) + b_g2_ref[...]

    gamma = gamma.reshape(K, NB3, C_OUT)
    delta = delta.reshape(K, NB3, C_OUT)
    m = jnp.max(gamma, axis=0)
    e = jnp.exp(gamma - m[None, :, :])
    s = jnp.sum(e, axis=0)
    acc = jnp.sum(e * delta, axis=0)
    out_ref[0] = alpha_ref[0] + acc / s


def _stage2(nbr, crdT, phiT, alphaT,
            W_t1p, b_t1, W_t2, b_t2, W_g1, b_g1, W_g2, b_g2):
    npts = nbr.shape[2]
    nblk = npts // NB3
    grid = (1, nblk)
    wspec = pl.BlockSpec((C_OUT, C_OUT), lambda b, i: (0, 0))
    bspec = pl.BlockSpec((1, C_OUT), lambda b, i: (0, 0))
    return pl.pallas_call(
        _stage2_body,
        grid=grid,
        in_specs=[
            pl.BlockSpec((1, K, NB3, TAB), lambda b, i: (b, 0, i, 0)),
            pl.BlockSpec((1, NB3, CPAD), lambda b, i: (b, i, 0)),
            pl.BlockSpec((1, NB3, C_OUT), lambda b, i: (b, i, 0)),
            pl.BlockSpec((1, NB3, C_OUT), lambda b, i: (b, i, 0)),
            pl.BlockSpec((C_OUT, CPAD), lambda b, i: (0, 0)), bspec,
            wspec, bspec, wspec, bspec, wspec, bspec,
        ],
        out_specs=pl.BlockSpec((1, NB3, C_OUT), lambda b, i: (b, i, 0)),
        out_shape=jax.ShapeDtypeStruct((1, npts, C_OUT), jnp.float32),
        compiler_params=pltpu.CompilerParams(
            dimension_semantics=("parallel", "parallel")),
    )(nbr, crdT, phiT, alphaT,
      W_t1p, b_t1.reshape(1, C_OUT), W_t2, b_t2.reshape(1, C_OUT),
      W_g1, b_g1.reshape(1, C_OUT), W_g2, b_g2.reshape(1, C_OUT))


def kernel(features, coords, W_phi, b_phi, W_psi, b_psi, W_alpha, b_alpha,
           W_g1, b_g1, W_g2, b_g2, W_t1, b_t1, W_t2, b_t2):
    fT = jnp.transpose(features, (0, 2, 1))
    crdT = jnp.pad(jnp.transpose(coords, (0, 2, 1)),
                   ((0, 0), (0, 0), (0, CPAD - C_COORD)))
    W_t1p = jnp.pad(W_t1, ((0, 0), (0, CPAD - C_COORD)))

    NCH = N // 2
    outs = []
    for b in range(B):
        phiT, alphaT, table = _stage1a(
            fT[b:b + 1], crdT[b:b + 1],
            W_phi, b_phi, W_psi, b_psi, W_alpha, b_alpha)
        chunks = []
        for c in range(N // NCH):
            lo, hi = c * NCH, (c + 1) * NCH
            idx_c = _stage1b(fT[b:b + 1], fT[b:b + 1, lo:hi])
            nbr = _gather_rows(table.reshape(N, TAB),
                               idx_c.reshape(1, K * NCH))
            chunks.append(_stage2(
                nbr.reshape(1, K, NCH, TAB), crdT[b:b + 1, lo:hi],
                phiT[:, lo:hi], alphaT[:, lo:hi],
                W_t1p, b_t1, W_t2, b_t2, W_g1, b_g1, W_g2, b_g2))
        outs.append(jnp.concatenate(chunks, axis=1))
    outT = jnp.concatenate(outs, axis=0)
    return jnp.transpose(outT, (0, 2, 1))


_gather_rows = _sc_gather


# speedup vs baseline: 1.0743x; 1.0617x over previous
import functools

import jax
import jax.numpy as jnp
from jax import lax
from jax.experimental import pallas as pl
from jax.experimental.pallas import tpu as pltpu
from jax.experimental.pallas import tpu_sc as plsc

B, C_IN, C_OUT, C_COORD, N, K = 2, 256, 256, 3, 4096, 16
NB1 = 256
NB3 = 256
CPAD = 128
HALF = C_OUT // 2
TAB = C_OUT // 2 + CPAD
BIGF = 3e38


def _pack16(x):
    bits = lax.bitcast_convert_type(x, jnp.int32)
    return lax.shift_right_arithmetic(bits + 0x8000, 16) & 0xFFFF


def _unpack_lo(w):
    return lax.bitcast_convert_type(lax.shift_left(w, 16), jnp.float32)


def _unpack_hi(w):
    return lax.bitcast_convert_type(w & jnp.int32(-65536), jnp.float32)


def _stage1a_body(fT_blk_ref, crd_ref, W_phi_ref, b_phi_ref,
                  W_psi_ref, b_psi_ref, W_alpha_ref, b_alpha_ref,
                  phiT_ref, alphaT_ref, tab_ref):
    f_blk = fT_blk_ref[0]
    dims = (((1,), (1,)), ((), ()))
    phiT_ref[0] = lax.dot_general(
        f_blk, W_phi_ref[...], dims,
        preferred_element_type=jnp.float32) + b_phi_ref[...]
    alphaT_ref[0] = lax.dot_general(
        f_blk, W_alpha_ref[...], dims,
        preferred_element_type=jnp.float32) + b_alpha_ref[...]
    psi = lax.dot_general(
        f_blk, W_psi_ref[...], dims,
        preferred_element_type=jnp.float32) + b_psi_ref[...]
    tab_ref[0, :, :HALF] = (_pack16(psi[:, :HALF])
                            | lax.shift_left(_pack16(psi[:, HALF:]), 16))
    tab_ref[0, :, HALF:] = _pack16(crd_ref[0])


def _stage1a(fT, crdT, W_phi, b_phi, W_psi, b_psi, W_alpha, b_alpha):
    nblk = N // NB1
    grid = (1, nblk)
    wspec = pl.BlockSpec((C_OUT, C_IN), lambda b, i: (0, 0))
    bspec = pl.BlockSpec((1, C_OUT), lambda b, i: (0, 0))
    outT = jax.ShapeDtypeStruct((1, N, C_OUT), jnp.float32)
    return pl.pallas_call(
        _stage1a_body,
        grid=grid,
        in_specs=[
            pl.BlockSpec((1, NB1, C_IN), lambda b, i: (b, i, 0)),
            pl.BlockSpec((1, NB1, CPAD), lambda b, i: (b, i, 0)),
            wspec, bspec, wspec, bspec, wspec, bspec,
        ],
        out_specs=[
            pl.BlockSpec((1, NB1, C_OUT), lambda b, i: (b, i, 0)),
            pl.BlockSpec((1, NB1, C_OUT), lambda b, i: (b, i, 0)),
            pl.BlockSpec((1, NB1, TAB), lambda b, i: (b, i, 0)),
        ],
        out_shape=[outT, outT,
                   jax.ShapeDtypeStruct((1, N, TAB), jnp.int32)],
        compiler_params=pltpu.CompilerParams(
            dimension_semantics=("parallel", "parallel")),
    )(fT, crdT, W_phi, b_phi.reshape(1, C_OUT), W_psi,
      b_psi.reshape(1, C_OUT), W_alpha, b_alpha.reshape(1, C_OUT))


def _stage1b_body(fT_all_ref, fT_blk_ref, idx_ref):
    f_all = fT_all_ref[0]
    f_blk = fT_blk_ref[0]
    dims = (((1,), (1,)), ((), ()))
    inner = lax.dot_general(f_all, f_blk, dims,
                            preferred_element_type=jnp.float32)
    sq = jnp.sum(f_all * f_all, axis=1, keepdims=True)
    dist = sq - (inner + inner)

    row_iota = lax.broadcasted_iota(jnp.int32, (N, NB1), 0)
    for k in range(K):
        sel = jnp.argmin(dist, axis=0).astype(jnp.int32)
        idx_ref[0, k] = sel
        dist = jnp.where(row_iota == sel[None, :], BIGF, dist)


def _stage1b(fT, fT_chunk):
    nch = fT_chunk.shape[1]
    nblk = nch // NB1
    grid = (1, nblk)
    return pl.pallas_call(
        _stage1b_body,
        grid=grid,
        in_specs=[
            pl.BlockSpec((1, N, C_IN), lambda b, i: (b, 0, 0)),
            pl.BlockSpec((1, NB1, C_IN), lambda b, i: (b, i, 0)),
        ],
        out_specs=pl.BlockSpec((1, K, NB1), lambda b, i: (b, 0, i)),
        out_shape=jax.ShapeDtypeStruct((1, K, nch), jnp.int32),
        compiler_params=pltpu.CompilerParams(
            dimension_semantics=("parallel", "parallel")),
    )(fT, fT_chunk)


_GW = 128


def _sc_gather(table, idx_flat):
    n_idx = idx_flat.shape[1]
    mesh = plsc.VectorSubcoreMesh(core_axis_name="c", subcore_axis_name="s")

    @functools.partial(
        pl.kernel,
        out_type=jax.ShapeDtypeStruct((n_idx, TAB), jnp.int32),
        mesh=mesh,
    )
    def gather_kernel(tab_hbm, i_hbm, out_hbm):
        def body(i_vmem, o_vmem):
            pltpu.sync_copy(tab_hbm.at[i_vmem.at[0]], o_vmem)

        pltpu.emit_pipeline(
            body,
            grid=(n_idx // _GW,),
            in_specs=[pl.BlockSpec((1, _GW), index_map=lambda i: (0, i))],
            out_specs=[pl.BlockSpec((_GW, TAB), index_map=lambda i: (i, 0))],
            core_axis_name=("c", "s"),
            dimension_semantics=(pltpu.PARALLEL,),
        )(i_hbm, out_hbm)

    return gather_kernel(table, idx_flat)


def _stage2_body(nbr_ref, crdc_ref, phi_ref, alpha_ref,
                 W_t1_ref, b_t1_ref, W_t2_ref, b_t2_ref,
                 W_g1_ref, b_g1_ref, W_g2_ref, b_g2_ref, out_ref):
    dims = (((1,), (1,)), ((), ()))
    M = K * NB3

    nbr = nbr_ref[0]
    psi_w = nbr[:, :, :HALF]
    psi_nbr = jnp.concatenate(
        [_unpack_lo(psi_w), _unpack_hi(psi_w)], axis=-1)
    c_nbr = _unpack_lo(nbr[:, :, HALF:])
    c_ctr = crdc_ref[0]

    bf = jnp.bfloat16
    cs = (c_ctr[None, :, :] - c_nbr).reshape(M, CPAD)
    h = lax.dot_general(cs.astype(bf), W_t1_ref[...].astype(bf), dims,
                        preferred_element_type=jnp.float32) + b_t1_ref[...]
    h = jnp.maximum(h, 0.0)
    delta = lax.dot_general(h.astype(bf), W_t2_ref[...].astype(bf), dims,
                            preferred_element_type=jnp.float32) + b_t2_ref[...]

    ginp = (phi_ref[0][None, :, :] - psi_nbr).reshape(M, C_OUT) + delta
    g = lax.dot_general(ginp.astype(bf), W_g1_ref[...].astype(bf), dims,
                        preferred_element_type=jnp.float32) + b_g1_ref[...]
    g = jnp.maximum(g, 0.0)
    gamma = lax.dot_general(g.astype(bf), W_g2_ref[...].astype(bf), dims,
                            preferred_element_type=jnp.float32) + b_g2_ref[...]

    gamma = gamma.reshape(K, NB3, C_OUT)
    delta = delta.reshape(K, NB3, C_OUT)
    m = jnp.max(gamma, axis=0)
    e = jnp.exp(gamma - m[None, :, :])
    s = jnp.sum(e, axis=0)
    acc = jnp.sum(e * delta, axis=0)
    out_ref[0] = alpha_ref[0] + acc / s


def _stage2(nbr, crdT, phiT, alphaT,
            W_t1p, b_t1, W_t2, b_t2, W_g1, b_g1, W_g2, b_g2):
    npts = nbr.shape[2]
    nblk = npts // NB3
    grid = (1, nblk)
    wspec = pl.BlockSpec((C_OUT, C_OUT), lambda b, i: (0, 0))
    bspec = pl.BlockSpec((1, C_OUT), lambda b, i: (0, 0))
    return pl.pallas_call(
        _stage2_body,
        grid=grid,
        in_specs=[
            pl.BlockSpec((1, K, NB3, TAB), lambda b, i: (b, 0, i, 0)),
            pl.BlockSpec((1, NB3, CPAD), lambda b, i: (b, i, 0)),
            pl.BlockSpec((1, NB3, C_OUT), lambda b, i: (b, i, 0)),
            pl.BlockSpec((1, NB3, C_OUT), lambda b, i: (b, i, 0)),
            pl.BlockSpec((C_OUT, CPAD), lambda b, i: (0, 0)), bspec,
            wspec, bspec, wspec, bspec, wspec, bspec,
        ],
        out_specs=pl.BlockSpec((1, NB3, C_OUT), lambda b, i: (b, i, 0)),
        out_shape=jax.ShapeDtypeStruct((1, npts, C_OUT), jnp.float32),
        compiler_params=pltpu.CompilerParams(
            dimension_semantics=("parallel", "parallel")),
    )(nbr, crdT, phiT, alphaT,
      W_t1p, b_t1.reshape(1, C_OUT), W_t2, b_t2.reshape(1, C_OUT),
      W_g1, b_g1.reshape(1, C_OUT), W_g2, b_g2.reshape(1, C_OUT))


def kernel(features, coords, W_phi, b_phi, W_psi, b_psi, W_alpha, b_alpha,
           W_g1, b_g1, W_g2, b_g2, W_t1, b_t1, W_t2, b_t2):
    fT = jnp.transpose(features, (0, 2, 1))
    crdT = jnp.pad(jnp.transpose(coords, (0, 2, 1)),
                   ((0, 0), (0, 0), (0, CPAD - C_COORD)))
    W_t1p = jnp.pad(W_t1, ((0, 0), (0, CPAD - C_COORD)))

    NCH = N
    outs = []
    for b in range(B):
        phiT, alphaT, table = _stage1a(
            fT[b:b + 1], crdT[b:b + 1],
            W_phi, b_phi, W_psi, b_psi, W_alpha, b_alpha)
        chunks = []
        for c in range(N // NCH):
            lo, hi = c * NCH, (c + 1) * NCH
            idx_c = _stage1b(fT[b:b + 1], fT[b:b + 1, lo:hi])
            nbr = _gather_rows(table.reshape(N, TAB),
                               idx_c.reshape(1, K * NCH))
            chunks.append(_stage2(
                nbr.reshape(1, K, NCH, TAB), crdT[b:b + 1, lo:hi],
                phiT[:, lo:hi], alphaT[:, lo:hi],
                W_t1p, b_t1, W_t2, b_t2, W_g1, b_g1, W_g2, b_g2))
        outs.append(jnp.concatenate(chunks, axis=1))
    outT = jnp.concatenate(outs, axis=0)
    return jnp.transpose(outT, (0, 2, 1))


_gather_rows = _sc_gather
